# baseline (speedup 1.0000x reference)
import functools

import jax
import jax.numpy as jnp
from jax import lax
from jax.experimental import pallas as pl
from jax.experimental.pallas import tpu as pltpu
from jax.experimental.pallas import tpu_sc as plsc

N = 10000
E = 320000
DIN = 128
HID = 64
LAT = 16

NPAD = 10240
ROWS_PER_TILE = NPAD // 16
CHUNK = 128
CHUNKS_PER_TILE = 79
EPT = CHUNK * CHUNKS_PER_TILE
EPAD = 32 * EPT
EPS = 1e-16


def _tc1_body(x_ref, w1_ref, asv_ref, adv_ref,
              xp_ref, asrc_ref, adst_ref, shift_ref):
    x = x_ref[...]
    xp = jnp.dot(x, w1_ref[...], preferred_element_type=jnp.float32)
    xp_ref[...] = xp
    a_s = jnp.sum(xp * asv_ref[...], axis=1, keepdims=True)
    a_d = jnp.sum(xp * adv_ref[...], axis=1, keepdims=True)
    asrc_ref[...] = jnp.pad(a_s, ((0, NPAD - N), (0, 0)))
    adst_ref[...] = jnp.pad(a_d, ((0, NPAD - N), (0, 0)))
    m = jnp.max(a_s) + jnp.max(a_d)
    shift_ref[...] = jnp.full((1, 128), m, dtype=jnp.float32)


def _tc2_body(s1_ref, den_ref, w2_ref, lat_ref):
    s = s1_ref[0] + s1_ref[1]
    d = den_ref[0] + den_ref[1]
    out1 = s / (d + EPS)
    h1 = jnp.where(out1 > 0.0, out1, jnp.exp(out1) - 1.0)
    lat_ref[...] = jnp.dot(h1, w2_ref[...], preferred_element_type=jnp.float32)


def _tc3_body(s2_ref, den_ref, w2_ref, w1_ref, rec_ref):
    s = s2_ref[0] + s2_ref[1]
    d = den_ref[0] + den_ref[1]
    p = s / (d + EPS)
    out3 = lax.dot_general(p, w2_ref[...], (((1,), (1,)), ((), ())),
                           preferred_element_type=jnp.float32)
    h3 = jnp.maximum(out3, 0.0)
    rec_ref[...] = lax.dot_general(h3, w1_ref[...], (((1,), (1,)), ((), ())),
                                   preferred_element_type=jnp.float32)


_SC_MESH = plsc.VectorSubcoreMesh(core_axis_name="c", subcore_axis_name="s")


def _zero16():
    return jnp.zeros((16,), jnp.float32)


@functools.partial(
    pl.kernel,
    out_type=[
        jax.ShapeDtypeStruct((2, NPAD, HID), jnp.float32),
        jax.ShapeDtypeStruct((2, NPAD), jnp.float32),
        jax.ShapeDtypeStruct((EPAD // CHUNK, CHUNK), jnp.float32),
    ],
    mesh=_SC_MESH,
    compiler_params=pltpu.CompilerParams(needs_layout_passes=False, use_tc_tiling_on_sc=False),
    scratch_types=[
        pltpu.VMEM((NPAD,), jnp.float32),
        pltpu.VMEM((NPAD,), jnp.float32),
        pltpu.VMEM((16,), jnp.float32),
        pltpu.VMEM((CHUNKS_PER_TILE, CHUNK), jnp.int32),
        pltpu.VMEM((CHUNKS_PER_TILE, CHUNK), jnp.int32),
        pltpu.VMEM((CHUNKS_PER_TILE, CHUNK), jnp.float32),
        pltpu.VMEM((CHUNK, HID), jnp.float32),
        pltpu.VMEM((CHUNK, HID), jnp.float32),
        pltpu.VMEM((ROWS_PER_TILE,), jnp.float32),
        pltpu.VMEM_SHARED((NPAD, HID), jnp.float32),
        pltpu.VMEM_SHARED((NPAD,), jnp.float32),
        pltpu.SemaphoreType.DMA,
        pltpu.SemaphoreType.DMA,
        pltpu.SemaphoreType.DMA,
        pltpu.SemaphoreType.DMA,
        pltpu.SemaphoreType.DMA,
    ],
)
def _sc_prop1(asrc_hbm, adst_hbm, shift_hbm, src2_hbm, dst2_hbm, xp_hbm,
              s1_out, den_out, ex_out,
              asrc_v, adst_v, shift_v, src2_v, dst2_v, ex2_v,
              rows_a, rows_b, zd_v, s1_sh, den_sh,
              sem_a, sem_b, sem_sa, sem_sb, sem_d):
    cid = lax.axis_index("c")
    sid = lax.axis_index("s")
    tid = cid * 16 + sid
    crow0 = pl.multiple_of(tid * CHUNKS_PER_TILE, 1)
    pltpu.sync_copy(asrc_hbm, asrc_v)
    pltpu.sync_copy(adst_hbm, adst_v)
    pltpu.sync_copy(shift_hbm, shift_v)
    pltpu.sync_copy(src2_hbm.at[pl.ds(crow0, CHUNKS_PER_TILE)], src2_v)
    pltpu.sync_copy(dst2_hbm.at[pl.ds(crow0, CHUNKS_PER_TILE)], dst2_v)

    def _zrows(i, c):
        for j in range(4):
            rows_a[i, pl.ds(j * 16, 16)] = _zero16()
        return c
    lax.fori_loop(0, CHUNK, _zrows, 0)

    def _zd(i, c):
        zd_v[pl.ds(pl.multiple_of(i * 16, 16), 16)] = _zero16()
        return c
    lax.fori_loop(0, ROWS_PER_TILE // 16, _zd, 0)

    r0 = pl.multiple_of(sid * ROWS_PER_TILE, ROWS_PER_TILE)
    for k in range(ROWS_PER_TILE // CHUNK):
        pltpu.sync_copy(rows_a, s1_sh.at[pl.ds(r0 + k * CHUNK, CHUNK)])
    pltpu.sync_copy(zd_v, den_sh.at[pl.ds(r0, ROWS_PER_TILE)])
    plsc.subcore_barrier()

    shv = shift_v[...]

    def _exchunk(c, carry):
        for g in range(8):
            si = src2_v[c, pl.ds(g * 16, 16)]
            di = dst2_v[c, pl.ds(g * 16, 16)]
            a = plsc.load_gather(asrc_v, [si]) + plsc.load_gather(adst_v, [di])
            a = jnp.where(a >= 0.0, a, 0.2 * a)
            ex2_v[c, pl.ds(g * 16, 16)] = jnp.exp(a - shv)
        return carry
    lax.fori_loop(0, CHUNKS_PER_TILE, _exchunk, 0)
    pltpu.sync_copy(ex2_v, ex_out.at[pl.ds(crow0, CHUNKS_PER_TILE)])

    def _scale(buf, c):
        def _mul(g, cc):
            exg = ex2_v[c, pl.ds(pl.multiple_of(g * 16, 16), 16)]
            for l in range(16):
                s = exg[l]
                e = g * 16 + l
                for j in range(4):
                    buf[e, pl.ds(j * 16, 16)] = buf[e, pl.ds(j * 16, 16)] * s
            return cc
        lax.fori_loop(0, CHUNK // 16, _mul, 0)

    pltpu.async_copy(xp_hbm.at[src2_v.at[0]], rows_a, sem_a)
    pltpu.async_copy(xp_hbm.at[src2_v.at[1]], rows_b, sem_b)

    def _pipe(i, carry):
        c0 = pl.multiple_of(i * 2, 2)
        c1 = c0 + 1
        pltpu.make_async_copy(xp_hbm.at[src2_v.at[c0]], rows_a, sem_a).wait()
        _scale(rows_a, c0)
        pltpu.async_copy(rows_a, s1_sh.at[dst2_v.at[c0]], sem_sa, add=True)
        pltpu.async_copy(ex2_v.at[c0], den_sh.at[dst2_v.at[c0]], sem_d,
                         add=True)
        pltpu.make_async_copy(xp_hbm.at[src2_v.at[c1]], rows_b, sem_b).wait()
        _scale(rows_b, c1)
        pltpu.async_copy(rows_b, s1_sh.at[dst2_v.at[c1]], sem_sb, add=True)
        pltpu.async_copy(ex2_v.at[c1], den_sh.at[dst2_v.at[c1]], sem_d,
                         add=True)
        pltpu.make_async_copy(rows_a, s1_sh.at[dst2_v.at[c0]], sem_sa).wait()
        pltpu.async_copy(xp_hbm.at[src2_v.at[c0 + 2]], rows_a, sem_a)

        @pl.when(i < (CHUNKS_PER_TILE - 1) // 2 - 1)
        def _():
            pltpu.make_async_copy(rows_b, s1_sh.at[dst2_v.at[c1]],
                                  sem_sb).wait()
            pltpu.async_copy(xp_hbm.at[src2_v.at[c1 + 2]], rows_b, sem_b)
        return carry
    lax.fori_loop(0, (CHUNKS_PER_TILE - 1) // 2, _pipe, 0)

    last = CHUNKS_PER_TILE - 1
    pltpu.make_async_copy(rows_b, s1_sh.at[dst2_v.at[last - 1]], sem_sb).wait()
    pltpu.make_async_copy(xp_hbm.at[src2_v.at[last]], rows_a, sem_a).wait()
    _scale(rows_a, last)
    pltpu.sync_copy(rows_a, s1_sh.at[dst2_v.at[last]], add=True)
    pltpu.async_copy(ex2_v.at[last], den_sh.at[dst2_v.at[last]], sem_d,
                     add=True)

    def _dend(c, carry):
        pltpu.make_async_copy(ex2_v.at[c], den_sh.at[dst2_v.at[c]],
                              sem_d).wait()
        return carry
    lax.fori_loop(0, CHUNKS_PER_TILE, _dend, 0)

    plsc.subcore_barrier()
    pltpu.sync_copy(s1_sh.at[pl.ds(r0, ROWS_PER_TILE)],
                    s1_out.at[cid, pl.ds(r0, ROWS_PER_TILE)])
    pltpu.sync_copy(den_sh.at[pl.ds(r0, ROWS_PER_TILE)],
                    den_out.at[cid, pl.ds(r0, ROWS_PER_TILE)])


@functools.partial(
    pl.kernel,
    out_type=[
        jax.ShapeDtypeStruct((2, NPAD, LAT), jnp.float32),
    ],
    mesh=_SC_MESH,
    compiler_params=pltpu.CompilerParams(needs_layout_passes=False, use_tc_tiling_on_sc=False),
    scratch_types=[
        pltpu.VMEM((CHUNKS_PER_TILE, CHUNK), jnp.int32),
        pltpu.VMEM((CHUNKS_PER_TILE, CHUNK), jnp.int32),
        pltpu.VMEM((CHUNKS_PER_TILE, CHUNK), jnp.float32),
        pltpu.VMEM((CHUNK, LAT), jnp.float32),
        pltpu.VMEM((CHUNK, LAT), jnp.float32),
        pltpu.VMEM_SHARED((NPAD, LAT), jnp.float32),
        pltpu.SemaphoreType.DMA,
        pltpu.SemaphoreType.DMA,
        pltpu.SemaphoreType.DMA,
        pltpu.SemaphoreType.DMA,
    ],
)
def _sc_prop2(src2_hbm, dst2_hbm, ex2_hbm, lat_hbm,
              s2_out,
              src2_v, dst2_v, ex2_v, rows_a, rows_b, s2_sh,
              sem_a, sem_b, sem_sa, sem_sb):
    cid = lax.axis_index("c")
    sid = lax.axis_index("s")
    tid = cid * 16 + sid
    crow0 = pl.multiple_of(tid * CHUNKS_PER_TILE, 1)
    pltpu.sync_copy(src2_hbm.at[pl.ds(crow0, CHUNKS_PER_TILE)], src2_v)
    pltpu.sync_copy(dst2_hbm.at[pl.ds(crow0, CHUNKS_PER_TILE)], dst2_v)
    pltpu.sync_copy(ex2_hbm.at[pl.ds(crow0, CHUNKS_PER_TILE)], ex2_v)

    def _zrows(i, c):
        rows_a[i, pl.ds(0, 16)] = _zero16()
        return c
    lax.fori_loop(0, CHUNK, _zrows, 0)

    r0 = pl.multiple_of(sid * ROWS_PER_TILE, ROWS_PER_TILE)
    for k in range(ROWS_PER_TILE // CHUNK):
        pltpu.sync_copy(rows_a, s2_sh.at[pl.ds(r0 + k * CHUNK, CHUNK)])
    plsc.subcore_barrier()

    def _scale(buf, c):
        def _mul(g, cc):
            exg = ex2_v[c, pl.ds(pl.multiple_of(g * 16, 16), 16)]
            for l in range(16):
                e = g * 16 + l
                buf[e, pl.ds(0, 16)] = buf[e, pl.ds(0, 16)] * exg[l]
            return cc
        lax.fori_loop(0, CHUNK // 16, _mul, 0)

    pltpu.async_copy(lat_hbm.at[src2_v.at[0]], rows_a, sem_a)
    pltpu.async_copy(lat_hbm.at[src2_v.at[1]], rows_b, sem_b)

    def _pipe(i, carry):
        c0 = pl.multiple_of(i * 2, 2)
        c1 = c0 + 1
        pltpu.make_async_copy(lat_hbm.at[src2_v.at[c0]], rows_a, sem_a).wait()
        _scale(rows_a, c0)
        pltpu.async_copy(rows_a, s2_sh.at[dst2_v.at[c0]], sem_sa, add=True)
        pltpu.make_async_copy(lat_hbm.at[src2_v.at[c1]], rows_b, sem_b).wait()
        _scale(rows_b, c1)
        pltpu.async_copy(rows_b, s2_sh.at[dst2_v.at[c1]], sem_sb, add=True)
        pltpu.make_async_copy(rows_a, s2_sh.at[dst2_v.at[c0]], sem_sa).wait()
        pltpu.async_copy(lat_hbm.at[src2_v.at[c0 + 2]], rows_a, sem_a)

        @pl.when(i < (CHUNKS_PER_TILE - 1) // 2 - 1)
        def _():
            pltpu.make_async_copy(rows_b, s2_sh.at[dst2_v.at[c1]],
                                  sem_sb).wait()
            pltpu.async_copy(lat_hbm.at[src2_v.at[c1 + 2]], rows_b, sem_b)
        return carry
    lax.fori_loop(0, (CHUNKS_PER_TILE - 1) // 2, _pipe, 0)

    last = CHUNKS_PER_TILE - 1
    pltpu.make_async_copy(rows_b, s2_sh.at[dst2_v.at[last - 1]], sem_sb).wait()
    pltpu.make_async_copy(lat_hbm.at[src2_v.at[last]], rows_a, sem_a).wait()
    _scale(rows_a, last)
    pltpu.sync_copy(rows_a, s2_sh.at[dst2_v.at[last]], add=True)

    plsc.subcore_barrier()
    pltpu.sync_copy(s2_sh.at[pl.ds(r0, ROWS_PER_TILE)],
                    s2_out.at[cid, pl.ds(r0, ROWS_PER_TILE)])


def kernel(x, edge_index, W1, W2, att_src1, att_dst1):
    asv = att_src1.reshape(1, HID)
    adv = att_dst1.reshape(1, HID)

    xp, asrc, adst, shift = pl.pallas_call(
        _tc1_body,
        out_shape=[
            jax.ShapeDtypeStruct((N, HID), jnp.float32),
            jax.ShapeDtypeStruct((NPAD, 1), jnp.float32),
            jax.ShapeDtypeStruct((NPAD, 1), jnp.float32),
            jax.ShapeDtypeStruct((1, 128), jnp.float32),
        ],
    )(x, W1, asv, adv)

    pad = EPAD - E
    src2 = jnp.concatenate([edge_index[0],
                            jnp.zeros((pad,), jnp.int32)]).reshape(-1, CHUNK)
    dst2 = jnp.concatenate([edge_index[1],
                            N + (jnp.arange(pad, dtype=jnp.int32) % 8)]
                           ).reshape(-1, CHUNK)

    s1p, denp, ex2 = _sc_prop1(asrc.reshape(NPAD), adst.reshape(NPAD),
                               shift[0, :16], src2, dst2, xp)

    den3 = denp.reshape(2, NPAD, 1)
    latent_full = pl.pallas_call(
        _tc2_body,
        out_shape=jax.ShapeDtypeStruct((NPAD, LAT), jnp.float32),
    )(s1p, den3, W2)

    s2p = _sc_prop2(src2, dst2, ex2, latent_full)
    if isinstance(s2p, (list, tuple)):
        s2p = s2p[0]

    recon_full = pl.pallas_call(
        _tc3_body,
        out_shape=jax.ShapeDtypeStruct((NPAD, DIN), jnp.float32),
    )(s2p, den3, W2, W1)

    return latent_full[:N], recon_full[:N]


# speedup vs baseline: 1.2931x; 1.2931x over previous
import functools

import jax
import jax.numpy as jnp
from jax import lax
from jax.experimental import pallas as pl
from jax.experimental.pallas import tpu as pltpu
from jax.experimental.pallas import tpu_sc as plsc

N = 10000
E = 320000
DIN = 128
HID = 64
LAT = 16

NPAD = 10240
ROWS_PER_TILE = NPAD // 16
CHUNK = 128
CHUNKS_PER_TILE = 79
EPT = CHUNK * CHUNKS_PER_TILE
EPAD = 32 * EPT
EPS = 1e-16


def _tc1_body(x_ref, w1_ref, asv_ref, adv_ref,
              xp_ref, asrc_ref, adst_ref, shift_ref):
    x = x_ref[...]
    xp = jnp.dot(x, w1_ref[...], preferred_element_type=jnp.float32)
    xp_ref[...] = xp
    a_s = jnp.sum(xp * asv_ref[...], axis=1, keepdims=True)
    a_d = jnp.sum(xp * adv_ref[...], axis=1, keepdims=True)
    asrc_ref[...] = jnp.pad(a_s, ((0, NPAD - N), (0, 0)))
    adst_ref[...] = jnp.pad(a_d, ((0, NPAD - N), (0, 0)))
    m = jnp.max(a_s) + jnp.max(a_d)
    shift_ref[...] = jnp.full((1, 128), m, dtype=jnp.float32)


def _tc2_body(s1_ref, den_ref, w2_ref, lat_ref):
    s = s1_ref[0] + s1_ref[1]
    d = den_ref[0] + den_ref[1]
    out1 = s / (d + EPS)
    h1 = jnp.where(out1 > 0.0, out1, jnp.exp(out1) - 1.0)
    lat_ref[...] = jnp.dot(h1, w2_ref[...],
                           preferred_element_type=jnp.float32)[:N]


def _tc3_body(s2_ref, den_ref, w2_ref, w1_ref, rec_ref):
    s = s2_ref[0] + s2_ref[1]
    d = den_ref[0] + den_ref[1]
    p = s / (d + EPS)
    out3 = lax.dot_general(p, w2_ref[...], (((1,), (1,)), ((), ())),
                           preferred_element_type=jnp.float32)
    h3 = jnp.maximum(out3, 0.0)
    rec_ref[...] = lax.dot_general(h3, w1_ref[...], (((1,), (1,)), ((), ())),
                                   preferred_element_type=jnp.float32)[:N]


_SC_MESH = plsc.VectorSubcoreMesh(core_axis_name="c", subcore_axis_name="s")


def _zero16():
    return jnp.zeros((16,), jnp.float32)


@functools.partial(
    pl.kernel,
    out_type=[
        jax.ShapeDtypeStruct((2, NPAD, HID), jnp.float32),
        jax.ShapeDtypeStruct((2, NPAD), jnp.float32),
        jax.ShapeDtypeStruct((EPAD // CHUNK, CHUNK), jnp.float32),
    ],
    mesh=_SC_MESH,
    compiler_params=pltpu.CompilerParams(needs_layout_passes=False, use_tc_tiling_on_sc=False),
    scratch_types=[
        pltpu.VMEM((NPAD,), jnp.float32),
        pltpu.VMEM((NPAD,), jnp.float32),
        pltpu.VMEM((16,), jnp.float32),
        pltpu.VMEM((CHUNKS_PER_TILE, CHUNK), jnp.int32),
        pltpu.VMEM((CHUNKS_PER_TILE, CHUNK), jnp.int32),
        pltpu.VMEM((CHUNKS_PER_TILE, CHUNK), jnp.float32),
        pltpu.VMEM((CHUNK, HID), jnp.float32),
        pltpu.VMEM((CHUNK, HID), jnp.float32),
        pltpu.VMEM((ROWS_PER_TILE,), jnp.float32),
        pltpu.VMEM_SHARED((NPAD, HID), jnp.float32),
        pltpu.VMEM_SHARED((NPAD,), jnp.float32),
        pltpu.SemaphoreType.DMA,
        pltpu.SemaphoreType.DMA,
        pltpu.SemaphoreType.DMA,
        pltpu.SemaphoreType.DMA,
        pltpu.SemaphoreType.DMA,
    ],
)
def _sc_prop1(asrc_hbm, adst_hbm, shift_hbm, src2_hbm, dst2_hbm, xp_hbm,
              s1_out, den_out, ex_out,
              asrc_v, adst_v, shift_v, src2_v, dst2_v, ex2_v,
              rows_a, rows_b, zd_v, s1_sh, den_sh,
              sem_a, sem_b, sem_sa, sem_sb, sem_d):
    cid = lax.axis_index("c")
    sid = lax.axis_index("s")
    tid = cid * 16 + sid
    crow0 = pl.multiple_of(tid * CHUNKS_PER_TILE, 1)
    pltpu.sync_copy(asrc_hbm, asrc_v)
    pltpu.sync_copy(adst_hbm, adst_v)
    pltpu.sync_copy(shift_hbm, shift_v)
    pltpu.sync_copy(src2_hbm.at[pl.ds(crow0, CHUNKS_PER_TILE)], src2_v)
    pltpu.sync_copy(dst2_hbm.at[pl.ds(crow0, CHUNKS_PER_TILE)], dst2_v)

    def _zrows(i, c):
        for j in range(4):
            rows_a[i, pl.ds(j * 16, 16)] = _zero16()
        return c
    lax.fori_loop(0, CHUNK, _zrows, 0)

    def _zd(i, c):
        zd_v[pl.ds(pl.multiple_of(i * 16, 16), 16)] = _zero16()
        return c
    lax.fori_loop(0, ROWS_PER_TILE // 16, _zd, 0)

    r0 = pl.multiple_of(sid * ROWS_PER_TILE, ROWS_PER_TILE)
    for k in range(ROWS_PER_TILE // CHUNK):
        pltpu.sync_copy(rows_a, s1_sh.at[pl.ds(r0 + k * CHUNK, CHUNK)])
    pltpu.sync_copy(zd_v, den_sh.at[pl.ds(r0, ROWS_PER_TILE)])
    plsc.subcore_barrier()

    shv = shift_v[...]

    def _exs(c):
        for g in range(8):
            si = src2_v[c, pl.ds(g * 16, 16)]
            di = dst2_v[c, pl.ds(g * 16, 16)]
            a = plsc.load_gather(asrc_v, [si]) + plsc.load_gather(adst_v, [di])
            a = jnp.where(a >= 0.0, a, 0.2 * a)
            ex2_v[c, pl.ds(g * 16, 16)] = jnp.exp(a - shv)

    def _scale(buf, c):
        def _mul(g, cc):
            exg = ex2_v[c, pl.ds(pl.multiple_of(g * 16, 16), 16)]
            for l in range(16):
                s = exg[l]
                e = g * 16 + l
                for j in range(4):
                    buf[e, pl.ds(j * 16, 16)] = buf[e, pl.ds(j * 16, 16)] * s
            return cc
        lax.fori_loop(0, CHUNK // 16, _mul, 0)

    pltpu.async_copy(xp_hbm.at[src2_v.at[0]], rows_a, sem_a)
    pltpu.async_copy(xp_hbm.at[src2_v.at[1]], rows_b, sem_b)

    def _pipe(i, carry):
        c0 = pl.multiple_of(i * 2, 2)
        c1 = c0 + 1
        _exs(c0)
        pltpu.make_async_copy(xp_hbm.at[src2_v.at[c0]], rows_a, sem_a).wait()
        _scale(rows_a, c0)
        pltpu.async_copy(rows_a, s1_sh.at[dst2_v.at[c0]], sem_sa, add=True)
        pltpu.async_copy(ex2_v.at[c0], den_sh.at[dst2_v.at[c0]], sem_d,
                         add=True)
        _exs(c1)
        pltpu.make_async_copy(xp_hbm.at[src2_v.at[c1]], rows_b, sem_b).wait()
        _scale(rows_b, c1)
        pltpu.async_copy(rows_b, s1_sh.at[dst2_v.at[c1]], sem_sb, add=True)
        pltpu.async_copy(ex2_v.at[c1], den_sh.at[dst2_v.at[c1]], sem_d,
                         add=True)
        pltpu.make_async_copy(rows_a, s1_sh.at[dst2_v.at[c0]], sem_sa).wait()
        pltpu.async_copy(xp_hbm.at[src2_v.at[c0 + 2]], rows_a, sem_a)

        @pl.when(i < (CHUNKS_PER_TILE - 1) // 2 - 1)
        def _():
            pltpu.make_async_copy(rows_b, s1_sh.at[dst2_v.at[c1]],
                                  sem_sb).wait()
            pltpu.async_copy(xp_hbm.at[src2_v.at[c1 + 2]], rows_b, sem_b)
        return carry
    lax.fori_loop(0, (CHUNKS_PER_TILE - 1) // 2, _pipe, 0)

    last = CHUNKS_PER_TILE - 1
    _exs(last)
    pltpu.make_async_copy(rows_b, s1_sh.at[dst2_v.at[last - 1]], sem_sb).wait()
    pltpu.make_async_copy(xp_hbm.at[src2_v.at[last]], rows_a, sem_a).wait()
    _scale(rows_a, last)
    pltpu.sync_copy(rows_a, s1_sh.at[dst2_v.at[last]], add=True)
    pltpu.async_copy(ex2_v.at[last], den_sh.at[dst2_v.at[last]], sem_d,
                     add=True)

    pltpu.sync_copy(ex2_v, ex_out.at[pl.ds(crow0, CHUNKS_PER_TILE)])

    def _dend(c, carry):
        pltpu.make_async_copy(ex2_v.at[c], den_sh.at[dst2_v.at[c]],
                              sem_d).wait()
        return carry
    lax.fori_loop(0, CHUNKS_PER_TILE, _dend, 0)

    plsc.subcore_barrier()
    pltpu.sync_copy(s1_sh.at[pl.ds(r0, ROWS_PER_TILE)],
                    s1_out.at[cid, pl.ds(r0, ROWS_PER_TILE)])
    pltpu.sync_copy(den_sh.at[pl.ds(r0, ROWS_PER_TILE)],
                    den_out.at[cid, pl.ds(r0, ROWS_PER_TILE)])


@functools.partial(
    pl.kernel,
    out_type=[
        jax.ShapeDtypeStruct((2, NPAD, LAT), jnp.float32),
    ],
    mesh=_SC_MESH,
    compiler_params=pltpu.CompilerParams(needs_layout_passes=False, use_tc_tiling_on_sc=False),
    scratch_types=[
        pltpu.VMEM((CHUNKS_PER_TILE, CHUNK), jnp.int32),
        pltpu.VMEM((CHUNKS_PER_TILE, CHUNK), jnp.int32),
        pltpu.VMEM((CHUNKS_PER_TILE, CHUNK), jnp.float32),
        pltpu.VMEM((CHUNK, LAT), jnp.float32),
        pltpu.VMEM((CHUNK, LAT), jnp.float32),
        pltpu.VMEM_SHARED((NPAD, LAT), jnp.float32),
        pltpu.SemaphoreType.DMA,
        pltpu.SemaphoreType.DMA,
        pltpu.SemaphoreType.DMA,
        pltpu.SemaphoreType.DMA,
    ],
)
def _sc_prop2(src2_hbm, dst2_hbm, ex2_hbm, lat_hbm,
              s2_out,
              src2_v, dst2_v, ex2_v, rows_a, rows_b, s2_sh,
              sem_a, sem_b, sem_sa, sem_sb):
    cid = lax.axis_index("c")
    sid = lax.axis_index("s")
    tid = cid * 16 + sid
    crow0 = pl.multiple_of(tid * CHUNKS_PER_TILE, 1)
    pltpu.sync_copy(src2_hbm.at[pl.ds(crow0, CHUNKS_PER_TILE)], src2_v)
    pltpu.sync_copy(dst2_hbm.at[pl.ds(crow0, CHUNKS_PER_TILE)], dst2_v)
    pltpu.sync_copy(ex2_hbm.at[pl.ds(crow0, CHUNKS_PER_TILE)], ex2_v)

    def _zrows(i, c):
        rows_a[i, pl.ds(0, 16)] = _zero16()
        return c
    lax.fori_loop(0, CHUNK, _zrows, 0)

    r0 = pl.multiple_of(sid * ROWS_PER_TILE, ROWS_PER_TILE)
    for k in range(ROWS_PER_TILE // CHUNK):
        pltpu.sync_copy(rows_a, s2_sh.at[pl.ds(r0 + k * CHUNK, CHUNK)])
    plsc.subcore_barrier()

    def _scale(buf, c):
        def _mul(g, cc):
            exg = ex2_v[c, pl.ds(pl.multiple_of(g * 16, 16), 16)]
            for l in range(16):
                e = g * 16 + l
                buf[e, pl.ds(0, 16)] = buf[e, pl.ds(0, 16)] * exg[l]
            return cc
        lax.fori_loop(0, CHUNK // 16, _mul, 0)

    pltpu.async_copy(lat_hbm.at[src2_v.at[0]], rows_a, sem_a)
    pltpu.async_copy(lat_hbm.at[src2_v.at[1]], rows_b, sem_b)

    def _pipe(i, carry):
        c0 = pl.multiple_of(i * 2, 2)
        c1 = c0 + 1
        pltpu.make_async_copy(lat_hbm.at[src2_v.at[c0]], rows_a, sem_a).wait()
        _scale(rows_a, c0)
        pltpu.async_copy(rows_a, s2_sh.at[dst2_v.at[c0]], sem_sa, add=True)
        pltpu.make_async_copy(lat_hbm.at[src2_v.at[c1]], rows_b, sem_b).wait()
        _scale(rows_b, c1)
        pltpu.async_copy(rows_b, s2_sh.at[dst2_v.at[c1]], sem_sb, add=True)
        pltpu.make_async_copy(rows_a, s2_sh.at[dst2_v.at[c0]], sem_sa).wait()
        pltpu.async_copy(lat_hbm.at[src2_v.at[c0 + 2]], rows_a, sem_a)

        @pl.when(i < (CHUNKS_PER_TILE - 1) // 2 - 1)
        def _():
            pltpu.make_async_copy(rows_b, s2_sh.at[dst2_v.at[c1]],
                                  sem_sb).wait()
            pltpu.async_copy(lat_hbm.at[src2_v.at[c1 + 2]], rows_b, sem_b)
        return carry
    lax.fori_loop(0, (CHUNKS_PER_TILE - 1) // 2, _pipe, 0)

    last = CHUNKS_PER_TILE - 1
    pltpu.make_async_copy(rows_b, s2_sh.at[dst2_v.at[last - 1]], sem_sb).wait()
    pltpu.make_async_copy(lat_hbm.at[src2_v.at[last]], rows_a, sem_a).wait()
    _scale(rows_a, last)
    pltpu.sync_copy(rows_a, s2_sh.at[dst2_v.at[last]], add=True)

    plsc.subcore_barrier()
    pltpu.sync_copy(s2_sh.at[pl.ds(r0, ROWS_PER_TILE)],
                    s2_out.at[cid, pl.ds(r0, ROWS_PER_TILE)])


def kernel(x, edge_index, W1, W2, att_src1, att_dst1):
    asv = att_src1.reshape(1, HID)
    adv = att_dst1.reshape(1, HID)

    xp, asrc, adst, shift = pl.pallas_call(
        _tc1_body,
        out_shape=[
            jax.ShapeDtypeStruct((N, HID), jnp.float32),
            jax.ShapeDtypeStruct((NPAD, 1), jnp.float32),
            jax.ShapeDtypeStruct((NPAD, 1), jnp.float32),
            jax.ShapeDtypeStruct((1, 128), jnp.float32),
        ],
    )(x, W1, asv, adv)

    pad = EPAD - E
    src2 = jnp.concatenate([edge_index[0],
                            jnp.zeros((pad,), jnp.int32)]).reshape(-1, CHUNK)
    dst2 = jnp.concatenate([edge_index[1],
                            N + (jnp.arange(pad, dtype=jnp.int32) % 8)]
                           ).reshape(-1, CHUNK)

    s1p, denp, ex2 = _sc_prop1(asrc.reshape(NPAD), adst.reshape(NPAD),
                               shift[0, :16], src2, dst2, xp)

    den3 = denp.reshape(2, NPAD, 1)
    latent = pl.pallas_call(
        _tc2_body,
        out_shape=jax.ShapeDtypeStruct((N, LAT), jnp.float32),
    )(s1p, den3, W2)

    s2p = _sc_prop2(src2, dst2, ex2, latent)
    if isinstance(s2p, (list, tuple)):
        s2p = s2p[0]

    recon = pl.pallas_call(
        _tc3_body,
        out_shape=jax.ShapeDtypeStruct((N, DIN), jnp.float32),
    )(s2p, den3, W2, W1)

    return latent, recon


# speedup vs baseline: 1.3458x; 1.0408x over previous
import functools

import jax
import jax.numpy as jnp
from jax import lax
from jax.experimental import pallas as pl
from jax.experimental.pallas import tpu as pltpu
from jax.experimental.pallas import tpu_sc as plsc

N = 10000
E = 320000
DIN = 128
HID = 64
LAT = 16

NPAD = 10240
ROWS_PER_TILE = NPAD // 16
CHUNK = 128
CHUNKS_PER_TILE = 79
TOTAL_CHUNKS = 32 * CHUNKS_PER_TILE
EPAD = TOTAL_CHUNKS * CHUNK
CA1, CB1 = 93, 65
CA2, CB2 = 85, 73
EPS = 1e-16


def _tc1_body(x_ref, w1_ref, asv_ref, adv_ref,
              xp_ref, asrc_ref, adst_ref, shift_ref):
    x = x_ref[...]
    xp = jnp.dot(x, w1_ref[...], preferred_element_type=jnp.float32)
    xp_ref[...] = xp
    a_s = jnp.sum(xp * asv_ref[...], axis=1, keepdims=True)
    a_d = jnp.sum(xp * adv_ref[...], axis=1, keepdims=True)
    asrc_ref[...] = jnp.pad(a_s, ((0, NPAD - N), (0, 0)))
    adst_ref[...] = jnp.pad(a_d, ((0, NPAD - N), (0, 0)))
    m = jnp.max(a_s) + jnp.max(a_d)
    shift_ref[...] = jnp.full((1, 128), m, dtype=jnp.float32)


def _tc2_body(s1_ref, den_ref, w2_ref, lat_ref):
    s = s1_ref[0] + s1_ref[1]
    d = den_ref[0] + den_ref[1]
    out1 = s / (d + EPS)
    h1 = jnp.where(out1 > 0.0, out1, jnp.exp(out1) - 1.0)
    lat_ref[...] = jnp.dot(h1, w2_ref[...],
                           preferred_element_type=jnp.float32)[:N]


def _tc3_body(s2_ref, den_ref, w2_ref, w1_ref, rec_ref):
    s = s2_ref[0] + s2_ref[1]
    d = den_ref[0] + den_ref[1]
    p = s / (d + EPS)
    out3 = lax.dot_general(p, w2_ref[...], (((1,), (1,)), ((), ())),
                           preferred_element_type=jnp.float32)
    h3 = jnp.maximum(out3, 0.0)
    rec_ref[...] = lax.dot_general(h3, w1_ref[...], (((1,), (1,)), ((), ())),
                                   preferred_element_type=jnp.float32)[:N]


_SC_MESH = plsc.VectorSubcoreMesh(core_axis_name="c", subcore_axis_name="s")


def _zero16():
    return jnp.zeros((16,), jnp.float32)


@functools.partial(
    pl.kernel,
    out_type=[
        jax.ShapeDtypeStruct((2, NPAD, HID), jnp.float32),
        jax.ShapeDtypeStruct((2, NPAD), jnp.float32),
        jax.ShapeDtypeStruct((EPAD // CHUNK, CHUNK), jnp.float32),
    ],
    mesh=_SC_MESH,
    compiler_params=pltpu.CompilerParams(needs_layout_passes=False, use_tc_tiling_on_sc=False),
    scratch_types=[
        pltpu.VMEM((NPAD,), jnp.float32),
        pltpu.VMEM((NPAD,), jnp.float32),
        pltpu.VMEM((16,), jnp.float32),
        pltpu.VMEM((CA1, CHUNK), jnp.int32),
        pltpu.VMEM((CA1, CHUNK), jnp.int32),
        pltpu.VMEM((CA1, CHUNK), jnp.float32),
        pltpu.VMEM((CHUNK, HID), jnp.float32),
        pltpu.VMEM((CHUNK, HID), jnp.float32),
        pltpu.VMEM((ROWS_PER_TILE,), jnp.float32),
        pltpu.VMEM_SHARED((NPAD, HID), jnp.float32),
        pltpu.VMEM_SHARED((NPAD,), jnp.float32),
        pltpu.SemaphoreType.DMA,
        pltpu.SemaphoreType.DMA,
        pltpu.SemaphoreType.DMA,
        pltpu.SemaphoreType.DMA,
        pltpu.SemaphoreType.DMA,
    ],
)
def _sc_prop1(asrc_hbm, adst_hbm, shift_hbm, src2_hbm, dst2_hbm, xp_hbm,
              s1_out, den_out, ex_out,
              asrc_v, adst_v, shift_v, src2_v, dst2_v, ex2_v,
              rows_a, rows_b, zd_v, s1_sh, den_sh,
              sem_a, sem_b, sem_sa, sem_sb, sem_d):
    cid = lax.axis_index("c")
    sid = lax.axis_index("s")
    pltpu.sync_copy(asrc_hbm, asrc_v)
    pltpu.sync_copy(adst_hbm, adst_v)
    pltpu.sync_copy(shift_hbm, shift_v)

    def _zrows(i, c):
        for j in range(4):
            rows_a[i, pl.ds(j * 16, 16)] = _zero16()
        return c
    lax.fori_loop(0, CHUNK, _zrows, 0)

    def _zd(i, c):
        zd_v[pl.ds(pl.multiple_of(i * 16, 16), 16)] = _zero16()
        return c
    lax.fori_loop(0, ROWS_PER_TILE // 16, _zd, 0)

    r0 = pl.multiple_of(sid * ROWS_PER_TILE, ROWS_PER_TILE)
    for k in range(ROWS_PER_TILE // CHUNK):
        pltpu.sync_copy(rows_a, s1_sh.at[pl.ds(r0 + k * CHUNK, CHUNK)])
    pltpu.sync_copy(zd_v, den_sh.at[pl.ds(r0, ROWS_PER_TILE)])
    plsc.subcore_barrier()

    shv = shift_v[...]

    def _exs(c):
        for g in range(8):
            si = src2_v[c, pl.ds(g * 16, 16)]
            di = dst2_v[c, pl.ds(g * 16, 16)]
            a = plsc.load_gather(asrc_v, [si]) + plsc.load_gather(adst_v, [di])
            a = jnp.where(a >= 0.0, a, 0.2 * a)
            ex2_v[c, pl.ds(g * 16, 16)] = jnp.exp(a - shv)

    def _scale(buf, c):
        def _mul(g, cc):
            exg = ex2_v[c, pl.ds(pl.multiple_of(g * 16, 16), 16)]
            for l in range(16):
                sc = exg[l]
                e = g * 16 + l
                for j in range(4):
                    buf[e, pl.ds(j * 16, 16)] = buf[e, pl.ds(j * 16, 16)] * sc
            return cc
        lax.fori_loop(0, CHUNK // 16, _mul, 0)

    def _run(nc, crow0):
        pltpu.sync_copy(src2_hbm.at[pl.ds(crow0, nc)],
                        src2_v.at[pl.ds(0, nc)])
        pltpu.sync_copy(dst2_hbm.at[pl.ds(crow0, nc)],
                        dst2_v.at[pl.ds(0, nc)])
        pltpu.async_copy(xp_hbm.at[src2_v.at[0]], rows_a, sem_a)
        pltpu.async_copy(xp_hbm.at[src2_v.at[1]], rows_b, sem_b)

        def _pipe(i, carry):
            c0 = pl.multiple_of(i * 2, 2)
            c1 = c0 + 1
            _exs(c0)
            pltpu.make_async_copy(xp_hbm.at[src2_v.at[c0]], rows_a,
                                  sem_a).wait()
            _scale(rows_a, c0)
            pltpu.async_copy(rows_a, s1_sh.at[dst2_v.at[c0]], sem_sa,
                             add=True)
            pltpu.async_copy(ex2_v.at[c0], den_sh.at[dst2_v.at[c0]], sem_d,
                             add=True)
            _exs(c1)
            pltpu.make_async_copy(xp_hbm.at[src2_v.at[c1]], rows_b,
                                  sem_b).wait()
            _scale(rows_b, c1)
            pltpu.async_copy(rows_b, s1_sh.at[dst2_v.at[c1]], sem_sb,
                             add=True)
            pltpu.async_copy(ex2_v.at[c1], den_sh.at[dst2_v.at[c1]], sem_d,
                             add=True)
            pltpu.make_async_copy(rows_a, s1_sh.at[dst2_v.at[c0]],
                                  sem_sa).wait()
            pltpu.async_copy(xp_hbm.at[src2_v.at[c0 + 2]], rows_a, sem_a)

            @pl.when(i < (nc - 1) // 2 - 1)
            def _():
                pltpu.make_async_copy(rows_b, s1_sh.at[dst2_v.at[c1]],
                                      sem_sb).wait()
                pltpu.async_copy(xp_hbm.at[src2_v.at[c1 + 2]], rows_b, sem_b)
            return carry
        lax.fori_loop(0, (nc - 1) // 2, _pipe, 0)

        last = nc - 1
        _exs(last)
        pltpu.make_async_copy(rows_b, s1_sh.at[dst2_v.at[last - 1]],
                              sem_sb).wait()
        pltpu.make_async_copy(xp_hbm.at[src2_v.at[last]], rows_a,
                              sem_a).wait()
        _scale(rows_a, last)
        pltpu.sync_copy(rows_a, s1_sh.at[dst2_v.at[last]], add=True)
        pltpu.async_copy(ex2_v.at[last], den_sh.at[dst2_v.at[last]], sem_d,
                         add=True)

        pltpu.sync_copy(ex2_v.at[pl.ds(0, nc)],
                        ex_out.at[pl.ds(crow0, nc)])

        def _dend(c, carry):
            pltpu.make_async_copy(ex2_v.at[c], den_sh.at[dst2_v.at[c]],
                                  sem_d).wait()
            return carry
        lax.fori_loop(0, nc, _dend, 0)

    @pl.when(cid == 0)
    def _():
        _run(CA1, pl.multiple_of(sid * CA1, 1))

    @pl.when(cid == 1)
    def _():
        _run(CB1, pl.multiple_of(16 * CA1 + sid * CB1, 1))

    plsc.subcore_barrier()
    pltpu.sync_copy(s1_sh.at[pl.ds(r0, ROWS_PER_TILE)],
                    s1_out.at[cid, pl.ds(r0, ROWS_PER_TILE)])
    pltpu.sync_copy(den_sh.at[pl.ds(r0, ROWS_PER_TILE)],
                    den_out.at[cid, pl.ds(r0, ROWS_PER_TILE)])


@functools.partial(
    pl.kernel,
    out_type=[
        jax.ShapeDtypeStruct((2, NPAD, LAT), jnp.float32),
    ],
    mesh=_SC_MESH,
    compiler_params=pltpu.CompilerParams(needs_layout_passes=False, use_tc_tiling_on_sc=False),
    scratch_types=[
        pltpu.VMEM((CA2, CHUNK), jnp.int32),
        pltpu.VMEM((CA2, CHUNK), jnp.int32),
        pltpu.VMEM((CA2, CHUNK), jnp.float32),
        pltpu.VMEM((CHUNK, LAT), jnp.float32),
        pltpu.VMEM((CHUNK, LAT), jnp.float32),
        pltpu.VMEM_SHARED((NPAD, LAT), jnp.float32),
        pltpu.SemaphoreType.DMA,
        pltpu.SemaphoreType.DMA,
        pltpu.SemaphoreType.DMA,
        pltpu.SemaphoreType.DMA,
    ],
)
def _sc_prop2(src2_hbm, dst2_hbm, ex2_hbm, lat_hbm,
              s2_out,
              src2_v, dst2_v, ex2_v, rows_a, rows_b, s2_sh,
              sem_a, sem_b, sem_sa, sem_sb):
    cid = lax.axis_index("c")
    sid = lax.axis_index("s")

    def _zrows(i, c):
        rows_a[i, pl.ds(0, 16)] = _zero16()
        return c
    lax.fori_loop(0, CHUNK, _zrows, 0)

    r0 = pl.multiple_of(sid * ROWS_PER_TILE, ROWS_PER_TILE)
    for k in range(ROWS_PER_TILE // CHUNK):
        pltpu.sync_copy(rows_a, s2_sh.at[pl.ds(r0 + k * CHUNK, CHUNK)])
    plsc.subcore_barrier()

    def _scale(buf, c):
        def _mul(g, cc):
            exg = ex2_v[c, pl.ds(pl.multiple_of(g * 16, 16), 16)]
            for l in range(16):
                e = g * 16 + l
                buf[e, pl.ds(0, 16)] = buf[e, pl.ds(0, 16)] * exg[l]
            return cc
        lax.fori_loop(0, CHUNK // 16, _mul, 0)

    def _run(nc, crow0):
        pltpu.sync_copy(src2_hbm.at[pl.ds(crow0, nc)],
                        src2_v.at[pl.ds(0, nc)])
        pltpu.sync_copy(dst2_hbm.at[pl.ds(crow0, nc)],
                        dst2_v.at[pl.ds(0, nc)])
        pltpu.sync_copy(ex2_hbm.at[pl.ds(crow0, nc)],
                        ex2_v.at[pl.ds(0, nc)])
        pltpu.async_copy(lat_hbm.at[src2_v.at[0]], rows_a, sem_a)
        pltpu.async_copy(lat_hbm.at[src2_v.at[1]], rows_b, sem_b)

        def _pipe(i, carry):
            c0 = pl.multiple_of(i * 2, 2)
            c1 = c0 + 1
            pltpu.make_async_copy(lat_hbm.at[src2_v.at[c0]], rows_a,
                                  sem_a).wait()
            _scale(rows_a, c0)
            pltpu.async_copy(rows_a, s2_sh.at[dst2_v.at[c0]], sem_sa,
                             add=True)
            pltpu.make_async_copy(lat_hbm.at[src2_v.at[c1]], rows_b,
                                  sem_b).wait()
            _scale(rows_b, c1)
            pltpu.async_copy(rows_b, s2_sh.at[dst2_v.at[c1]], sem_sb,
                             add=True)
            pltpu.make_async_copy(rows_a, s2_sh.at[dst2_v.at[c0]],
                                  sem_sa).wait()
            pltpu.async_copy(lat_hbm.at[src2_v.at[c0 + 2]], rows_a, sem_a)

            @pl.when(i < (nc - 1) // 2 - 1)
            def _():
                pltpu.make_async_copy(rows_b, s2_sh.at[dst2_v.at[c1]],
                                      sem_sb).wait()
                pltpu.async_copy(lat_hbm.at[src2_v.at[c1 + 2]], rows_b, sem_b)
            return carry
        lax.fori_loop(0, (nc - 1) // 2, _pipe, 0)

        last = nc - 1
        pltpu.make_async_copy(rows_b, s2_sh.at[dst2_v.at[last - 1]],
                              sem_sb).wait()
        pltpu.make_async_copy(lat_hbm.at[src2_v.at[last]], rows_a,
                              sem_a).wait()
        _scale(rows_a, last)
        pltpu.sync_copy(rows_a, s2_sh.at[dst2_v.at[last]], add=True)

    @pl.when(cid == 0)
    def _():
        _run(CA2, pl.multiple_of(sid * CA2, 1))

    @pl.when(cid == 1)
    def _():
        _run(CB2, pl.multiple_of(16 * CA2 + sid * CB2, 1))

    plsc.subcore_barrier()
    pltpu.sync_copy(s2_sh.at[pl.ds(r0, ROWS_PER_TILE)],
                    s2_out.at[cid, pl.ds(r0, ROWS_PER_TILE)])


def kernel(x, edge_index, W1, W2, att_src1, att_dst1):
    asv = att_src1.reshape(1, HID)
    adv = att_dst1.reshape(1, HID)

    xp, asrc, adst, shift = pl.pallas_call(
        _tc1_body,
        out_shape=[
            jax.ShapeDtypeStruct((N, HID), jnp.float32),
            jax.ShapeDtypeStruct((NPAD, 1), jnp.float32),
            jax.ShapeDtypeStruct((NPAD, 1), jnp.float32),
            jax.ShapeDtypeStruct((1, 128), jnp.float32),
        ],
    )(x, W1, asv, adv)

    pad = EPAD - E
    src2 = jnp.concatenate([edge_index[0],
                            jnp.zeros((pad,), jnp.int32)]).reshape(-1, CHUNK)
    dst2 = jnp.concatenate([edge_index[1],
                            N + (jnp.arange(pad, dtype=jnp.int32) % 8)]
                           ).reshape(-1, CHUNK)

    s1p, denp, ex2 = _sc_prop1(asrc.reshape(NPAD), adst.reshape(NPAD),
                               shift[0, :16], src2, dst2, xp)

    den3 = denp.reshape(2, NPAD, 1)
    latent = pl.pallas_call(
        _tc2_body,
        out_shape=jax.ShapeDtypeStruct((N, LAT), jnp.float32),
    )(s1p, den3, W2)

    s2p = _sc_prop2(src2, dst2, ex2, latent)
    if isinstance(s2p, (list, tuple)):
        s2p = s2p[0]

    recon = pl.pallas_call(
        _tc3_body,
        out_shape=jax.ShapeDtypeStruct((N, DIN), jnp.float32),
    )(s2p, den3, W2, W1)

    return latent, recon


# speedup vs baseline: 1.3603x; 1.0108x over previous
import functools

import jax
import jax.numpy as jnp
from jax import lax
from jax.experimental import pallas as pl
from jax.experimental.pallas import tpu as pltpu
from jax.experimental.pallas import tpu_sc as plsc

N = 10000
E = 320000
DIN = 128
HID = 64
LAT = 16

NPAD = 10240
ROWS_PER_TILE = NPAD // 16
CHUNK = 128
CHUNKS_PER_TILE = 79
TOTAL_CHUNKS = 32 * CHUNKS_PER_TILE
EPAD = TOTAL_CHUNKS * CHUNK
CA1, CB1 = 93, 65
CA2, CB2 = 85, 73
EPS = 1e-16


def _tc1_body(x_ref, w1_ref, asv_ref, adv_ref,
              xp_ref, asrc_ref, adst_ref, shift_ref):
    x = x_ref[...]
    xp = jnp.dot(x, w1_ref[...], preferred_element_type=jnp.float32)
    xp_ref[...] = xp
    a_s = jnp.sum(xp * asv_ref[...], axis=1, keepdims=True)
    a_d = jnp.sum(xp * adv_ref[...], axis=1, keepdims=True)
    asrc_ref[...] = jnp.pad(a_s, ((0, NPAD - N), (0, 0)))
    adst_ref[...] = jnp.pad(a_d, ((0, NPAD - N), (0, 0)))
    m = jnp.max(a_s) + jnp.max(a_d)
    shift_ref[...] = jnp.full((1, 128), m, dtype=jnp.float32)


def _tc2_body(s1_ref, den_ref, w2_ref, lat_ref):
    s = s1_ref[0] + s1_ref[1]
    d = den_ref[0] + den_ref[1]
    out1 = s / (d + EPS)
    h1 = jnp.where(out1 > 0.0, out1, jnp.exp(out1) - 1.0)
    lat_ref[...] = jnp.dot(h1, w2_ref[...],
                           preferred_element_type=jnp.float32)[:N]


def _tc3_body(s2_ref, den_ref, w2_ref, w1_ref, rec_ref):
    s = s2_ref[0] + s2_ref[1]
    d = den_ref[0] + den_ref[1]
    p = s / (d + EPS)
    out3 = lax.dot_general(p, w2_ref[...], (((1,), (1,)), ((), ())),
                           preferred_element_type=jnp.float32)
    h3 = jnp.maximum(out3, 0.0)
    rec_ref[...] = lax.dot_general(h3, w1_ref[...], (((1,), (1,)), ((), ())),
                                   preferred_element_type=jnp.float32)[:N]


_SC_MESH = plsc.VectorSubcoreMesh(core_axis_name="c", subcore_axis_name="s")


def _zero16():
    return jnp.zeros((16,), jnp.float32)


@functools.partial(
    pl.kernel,
    out_type=[
        jax.ShapeDtypeStruct((2, NPAD, HID), jnp.float32),
        jax.ShapeDtypeStruct((2, NPAD), jnp.float32),
        jax.ShapeDtypeStruct((EPAD // CHUNK, CHUNK), jnp.float32),
    ],
    mesh=_SC_MESH,
    compiler_params=pltpu.CompilerParams(needs_layout_passes=False, use_tc_tiling_on_sc=False),
    scratch_types=[
        pltpu.VMEM((NPAD,), jnp.float32),
        pltpu.VMEM((NPAD,), jnp.float32),
        pltpu.VMEM((16,), jnp.float32),
        pltpu.VMEM((CA1, CHUNK), jnp.int32),
        pltpu.VMEM((CA1, CHUNK), jnp.int32),
        pltpu.VMEM((CA1, CHUNK), jnp.float32),
        pltpu.VMEM((CHUNK, HID), jnp.float32),
        pltpu.VMEM((CHUNK, HID), jnp.float32),
        pltpu.VMEM((ROWS_PER_TILE,), jnp.float32),
        pltpu.VMEM_SHARED((NPAD, HID), jnp.float32),
        pltpu.VMEM_SHARED((NPAD,), jnp.float32),
        pltpu.VMEM_SHARED((NPAD,), jnp.float32),
        pltpu.VMEM_SHARED((NPAD,), jnp.float32),
        pltpu.SemaphoreType.DMA,
        pltpu.SemaphoreType.DMA,
        pltpu.SemaphoreType.DMA,
        pltpu.SemaphoreType.DMA,
        pltpu.SemaphoreType.DMA,
    ],
)
def _sc_prop1(asrc_hbm, adst_hbm, shift_hbm, src2_hbm, dst2_hbm, xp_hbm,
              s1_out, den_out, ex_out,
              asrc_v, adst_v, shift_v, src2_v, dst2_v, ex2_v,
              rows_a, rows_b, zd_v, s1_sh, den_sh, as_sh, ad_sh,
              sem_a, sem_b, sem_sa, sem_sb, sem_d):
    cid = lax.axis_index("c")
    sid = lax.axis_index("s")

    @pl.when(sid == 0)
    def _():
        pltpu.sync_copy(asrc_hbm, as_sh)
        pltpu.sync_copy(adst_hbm, ad_sh)
    pltpu.sync_copy(shift_hbm, shift_v)

    def _zrows(i, c):
        for j in range(4):
            rows_a[i, pl.ds(j * 16, 16)] = _zero16()
        return c
    lax.fori_loop(0, CHUNK, _zrows, 0)

    def _zd(i, c):
        zd_v[pl.ds(pl.multiple_of(i * 16, 16), 16)] = _zero16()
        return c
    lax.fori_loop(0, ROWS_PER_TILE // 16, _zd, 0)

    r0 = pl.multiple_of(sid * ROWS_PER_TILE, ROWS_PER_TILE)
    for k in range(ROWS_PER_TILE // CHUNK):
        pltpu.sync_copy(rows_a, s1_sh.at[pl.ds(r0 + k * CHUNK, CHUNK)])
    pltpu.sync_copy(zd_v, den_sh.at[pl.ds(r0, ROWS_PER_TILE)])
    plsc.subcore_barrier()
    pltpu.sync_copy(as_sh, asrc_v)
    pltpu.sync_copy(ad_sh, adst_v)

    shv = shift_v[...]

    def _exs(c):
        for g in range(8):
            si = src2_v[c, pl.ds(g * 16, 16)]
            di = dst2_v[c, pl.ds(g * 16, 16)]
            a = plsc.load_gather(asrc_v, [si]) + plsc.load_gather(adst_v, [di])
            a = jnp.where(a >= 0.0, a, 0.2 * a)
            ex2_v[c, pl.ds(g * 16, 16)] = jnp.exp(a - shv)

    def _scale(buf, c):
        def _mul(g, cc):
            exg = ex2_v[c, pl.ds(pl.multiple_of(g * 16, 16), 16)]
            for l in range(16):
                sc = exg[l]
                e = g * 16 + l
                for j in range(4):
                    buf[e, pl.ds(j * 16, 16)] = buf[e, pl.ds(j * 16, 16)] * sc
            return cc
        lax.fori_loop(0, CHUNK // 16, _mul, 0)

    def _run(nc, crow0):
        pltpu.sync_copy(src2_hbm.at[pl.ds(crow0, nc)],
                        src2_v.at[pl.ds(0, nc)])
        pltpu.sync_copy(dst2_hbm.at[pl.ds(crow0, nc)],
                        dst2_v.at[pl.ds(0, nc)])
        pltpu.async_copy(xp_hbm.at[src2_v.at[0]], rows_a, sem_a)
        pltpu.async_copy(xp_hbm.at[src2_v.at[1]], rows_b, sem_b)

        def _pipe(i, carry):
            c0 = pl.multiple_of(i * 2, 2)
            c1 = c0 + 1
            _exs(c0)
            pltpu.make_async_copy(xp_hbm.at[src2_v.at[c0]], rows_a,
                                  sem_a).wait()
            _scale(rows_a, c0)
            pltpu.async_copy(rows_a, s1_sh.at[dst2_v.at[c0]], sem_sa,
                             add=True)
            pltpu.async_copy(ex2_v.at[c0], den_sh.at[dst2_v.at[c0]], sem_d,
                             add=True)
            _exs(c1)
            pltpu.make_async_copy(xp_hbm.at[src2_v.at[c1]], rows_b,
                                  sem_b).wait()
            _scale(rows_b, c1)
            pltpu.async_copy(rows_b, s1_sh.at[dst2_v.at[c1]], sem_sb,
                             add=True)
            pltpu.async_copy(ex2_v.at[c1], den_sh.at[dst2_v.at[c1]], sem_d,
                             add=True)
            pltpu.make_async_copy(rows_a, s1_sh.at[dst2_v.at[c0]],
                                  sem_sa).wait()
            pltpu.async_copy(xp_hbm.at[src2_v.at[c0 + 2]], rows_a, sem_a)

            @pl.when(i < (nc - 1) // 2 - 1)
            def _():
                pltpu.make_async_copy(rows_b, s1_sh.at[dst2_v.at[c1]],
                                      sem_sb).wait()
                pltpu.async_copy(xp_hbm.at[src2_v.at[c1 + 2]], rows_b, sem_b)
            return carry
        lax.fori_loop(0, (nc - 1) // 2, _pipe, 0)

        last = nc - 1
        _exs(last)
        pltpu.make_async_copy(rows_b, s1_sh.at[dst2_v.at[last - 1]],
                              sem_sb).wait()
        pltpu.make_async_copy(xp_hbm.at[src2_v.at[last]], rows_a,
                              sem_a).wait()
        _scale(rows_a, last)
        pltpu.sync_copy(rows_a, s1_sh.at[dst2_v.at[last]], add=True)
        pltpu.async_copy(ex2_v.at[last], den_sh.at[dst2_v.at[last]], sem_d,
                         add=True)

        pltpu.sync_copy(ex2_v.at[pl.ds(0, nc)],
                        ex_out.at[pl.ds(crow0, nc)])

        def _dend(c, carry):
            pltpu.make_async_copy(ex2_v.at[c], den_sh.at[dst2_v.at[c]],
                                  sem_d).wait()
            return carry
        lax.fori_loop(0, nc, _dend, 0)

    @pl.when(cid == 0)
    def _():
        _run(CA1, pl.multiple_of(sid * CA1, 1))

    @pl.when(cid == 1)
    def _():
        _run(CB1, pl.multiple_of(16 * CA1 + sid * CB1, 1))

    plsc.subcore_barrier()
    pltpu.sync_copy(s1_sh.at[pl.ds(r0, ROWS_PER_TILE)],
                    s1_out.at[cid, pl.ds(r0, ROWS_PER_TILE)])
    pltpu.sync_copy(den_sh.at[pl.ds(r0, ROWS_PER_TILE)],
                    den_out.at[cid, pl.ds(r0, ROWS_PER_TILE)])


@functools.partial(
    pl.kernel,
    out_type=[
        jax.ShapeDtypeStruct((2, NPAD, LAT), jnp.float32),
    ],
    mesh=_SC_MESH,
    compiler_params=pltpu.CompilerParams(needs_layout_passes=False, use_tc_tiling_on_sc=False),
    scratch_types=[
        pltpu.VMEM((CA2, CHUNK), jnp.int32),
        pltpu.VMEM((CA2, CHUNK), jnp.int32),
        pltpu.VMEM((CA2, CHUNK), jnp.float32),
        pltpu.VMEM((CHUNK, LAT), jnp.float32),
        pltpu.VMEM((CHUNK, LAT), jnp.float32),
        pltpu.VMEM_SHARED((NPAD, LAT), jnp.float32),
        pltpu.SemaphoreType.DMA,
        pltpu.SemaphoreType.DMA,
        pltpu.SemaphoreType.DMA,
        pltpu.SemaphoreType.DMA,
    ],
)
def _sc_prop2(src2_hbm, dst2_hbm, ex2_hbm, lat_hbm,
              s2_out,
              src2_v, dst2_v, ex2_v, rows_a, rows_b, s2_sh,
              sem_a, sem_b, sem_sa, sem_sb):
    cid = lax.axis_index("c")
    sid = lax.axis_index("s")

    def _zrows(i, c):
        rows_a[i, pl.ds(0, 16)] = _zero16()
        return c
    lax.fori_loop(0, CHUNK, _zrows, 0)

    r0 = pl.multiple_of(sid * ROWS_PER_TILE, ROWS_PER_TILE)
    for k in range(ROWS_PER_TILE // CHUNK):
        pltpu.sync_copy(rows_a, s2_sh.at[pl.ds(r0 + k * CHUNK, CHUNK)])
    plsc.subcore_barrier()

    def _scale(buf, c):
        def _mul(g, cc):
            exg = ex2_v[c, pl.ds(pl.multiple_of(g * 16, 16), 16)]
            for l in range(16):
                e = g * 16 + l
                buf[e, pl.ds(0, 16)] = buf[e, pl.ds(0, 16)] * exg[l]
            return cc
        lax.fori_loop(0, CHUNK // 16, _mul, 0)

    def _run(nc, crow0):
        pltpu.sync_copy(src2_hbm.at[pl.ds(crow0, nc)],
                        src2_v.at[pl.ds(0, nc)])
        pltpu.sync_copy(dst2_hbm.at[pl.ds(crow0, nc)],
                        dst2_v.at[pl.ds(0, nc)])
        pltpu.sync_copy(ex2_hbm.at[pl.ds(crow0, nc)],
                        ex2_v.at[pl.ds(0, nc)])
        pltpu.async_copy(lat_hbm.at[src2_v.at[0]], rows_a, sem_a)
        pltpu.async_copy(lat_hbm.at[src2_v.at[1]], rows_b, sem_b)

        def _pipe(i, carry):
            c0 = pl.multiple_of(i * 2, 2)
            c1 = c0 + 1
            pltpu.make_async_copy(lat_hbm.at[src2_v.at[c0]], rows_a,
                                  sem_a).wait()
            _scale(rows_a, c0)
            pltpu.async_copy(rows_a, s2_sh.at[dst2_v.at[c0]], sem_sa,
                             add=True)
            pltpu.make_async_copy(lat_hbm.at[src2_v.at[c1]], rows_b,
                                  sem_b).wait()
            _scale(rows_b, c1)
            pltpu.async_copy(rows_b, s2_sh.at[dst2_v.at[c1]], sem_sb,
                             add=True)
            pltpu.make_async_copy(rows_a, s2_sh.at[dst2_v.at[c0]],
                                  sem_sa).wait()
            pltpu.async_copy(lat_hbm.at[src2_v.at[c0 + 2]], rows_a, sem_a)

            @pl.when(i < (nc - 1) // 2 - 1)
            def _():
                pltpu.make_async_copy(rows_b, s2_sh.at[dst2_v.at[c1]],
                                      sem_sb).wait()
                pltpu.async_copy(lat_hbm.at[src2_v.at[c1 + 2]], rows_b, sem_b)
            return carry
        lax.fori_loop(0, (nc - 1) // 2, _pipe, 0)

        last = nc - 1
        pltpu.make_async_copy(rows_b, s2_sh.at[dst2_v.at[last - 1]],
                              sem_sb).wait()
        pltpu.make_async_copy(lat_hbm.at[src2_v.at[last]], rows_a,
                              sem_a).wait()
        _scale(rows_a, last)
        pltpu.sync_copy(rows_a, s2_sh.at[dst2_v.at[last]], add=True)

    @pl.when(cid == 0)
    def _():
        _run(CA2, pl.multiple_of(sid * CA2, 1))

    @pl.when(cid == 1)
    def _():
        _run(CB2, pl.multiple_of(16 * CA2 + sid * CB2, 1))

    plsc.subcore_barrier()
    pltpu.sync_copy(s2_sh.at[pl.ds(r0, ROWS_PER_TILE)],
                    s2_out.at[cid, pl.ds(r0, ROWS_PER_TILE)])


def kernel(x, edge_index, W1, W2, att_src1, att_dst1):
    asv = att_src1.reshape(1, HID)
    adv = att_dst1.reshape(1, HID)

    xp, asrc, adst, shift = pl.pallas_call(
        _tc1_body,
        out_shape=[
            jax.ShapeDtypeStruct((N, HID), jnp.float32),
            jax.ShapeDtypeStruct((NPAD, 1), jnp.float32),
            jax.ShapeDtypeStruct((NPAD, 1), jnp.float32),
            jax.ShapeDtypeStruct((1, 128), jnp.float32),
        ],
    )(x, W1, asv, adv)

    pad = EPAD - E
    src2 = jnp.concatenate([edge_index[0],
                            jnp.zeros((pad,), jnp.int32)]).reshape(-1, CHUNK)
    dst2 = jnp.concatenate([edge_index[1],
                            N + (jnp.arange(pad, dtype=jnp.int32) % 8)]
                           ).reshape(-1, CHUNK)

    s1p, denp, ex2 = _sc_prop1(asrc.reshape(NPAD), adst.reshape(NPAD),
                               shift[0, :16], src2, dst2, xp)

    den3 = denp.reshape(2, NPAD, 1)
    latent = pl.pallas_call(
        _tc2_body,
        out_shape=jax.ShapeDtypeStruct((N, LAT), jnp.float32),
    )(s1p, den3, W2)

    s2p = _sc_prop2(src2, dst2, ex2, latent)
    if isinstance(s2p, (list, tuple)):
        s2p = s2p[0]

    recon = pl.pallas_call(
        _tc3_body,
        out_shape=jax.ShapeDtypeStruct((N, DIN), jnp.float32),
    )(s2p, den3, W2, W1)

    return latent, recon


# speedup vs baseline: 1.3864x; 1.0192x over previous
import functools

import jax
import jax.numpy as jnp
from jax import lax
from jax.experimental import pallas as pl
from jax.experimental.pallas import tpu as pltpu
from jax.experimental.pallas import tpu_sc as plsc

N = 10000
E = 320000
DIN = 128
HID = 64
LAT = 16

NPAD = 10240
ROWS_PER_TILE = NPAD // 16
CHUNK = 128
CHUNKS_PER_TILE = 79
TOTAL_CHUNKS = 32 * CHUNKS_PER_TILE
EPAD = TOTAL_CHUNKS * CHUNK
CA1, CB1 = 101, 57
CA2, CB2 = 85, 73
EPS = 1e-16


def _tc1_body(x_ref, w1_ref, asv_ref, adv_ref,
              xp_ref, asrc_ref, adst_ref, shift_ref):
    x = x_ref[...]
    xp = jnp.dot(x, w1_ref[...], preferred_element_type=jnp.float32)
    xp_ref[...] = xp
    a_s = jnp.sum(xp * asv_ref[...], axis=1, keepdims=True)
    a_d = jnp.sum(xp * adv_ref[...], axis=1, keepdims=True)
    asrc_ref[...] = jnp.pad(a_s, ((0, NPAD - N), (0, 0)))
    adst_ref[...] = jnp.pad(a_d, ((0, NPAD - N), (0, 0)))
    m = jnp.max(a_s) + jnp.max(a_d)
    shift_ref[...] = jnp.full((1, 128), m, dtype=jnp.float32)


def _tc2_body(s1_ref, den_ref, w2_ref, lat_ref):
    s = s1_ref[0] + s1_ref[1]
    d = den_ref[0] + den_ref[1]
    out1 = s / (d + EPS)
    h1 = jnp.where(out1 > 0.0, out1, jnp.exp(out1) - 1.0)
    lat_ref[...] = jnp.dot(h1, w2_ref[...],
                           preferred_element_type=jnp.float32)[:N]


def _tc3_body(s2_ref, den_ref, w2_ref, w1_ref, rec_ref):
    s = s2_ref[0] + s2_ref[1]
    d = den_ref[0] + den_ref[1]
    p = s / (d + EPS)
    out3 = lax.dot_general(p, w2_ref[...], (((1,), (1,)), ((), ())),
                           preferred_element_type=jnp.float32)
    h3 = jnp.maximum(out3, 0.0)
    rec_ref[...] = lax.dot_general(h3, w1_ref[...], (((1,), (1,)), ((), ())),
                                   preferred_element_type=jnp.float32)[:N]


_SC_MESH = plsc.VectorSubcoreMesh(core_axis_name="c", subcore_axis_name="s")


def _zero16():
    return jnp.zeros((16,), jnp.float32)


@functools.partial(
    pl.kernel,
    out_type=[
        jax.ShapeDtypeStruct((2, NPAD, HID), jnp.float32),
        jax.ShapeDtypeStruct((2, NPAD), jnp.float32),
        jax.ShapeDtypeStruct((EPAD // CHUNK, CHUNK), jnp.float32),
    ],
    mesh=_SC_MESH,
    compiler_params=pltpu.CompilerParams(needs_layout_passes=False, use_tc_tiling_on_sc=False),
    scratch_types=[
        pltpu.VMEM((NPAD,), jnp.float32),
        pltpu.VMEM((NPAD,), jnp.float32),
        pltpu.VMEM((16,), jnp.float32),
        pltpu.VMEM((CA1, CHUNK), jnp.int32),
        pltpu.VMEM((CA1, CHUNK), jnp.int32),
        pltpu.VMEM((CA1, CHUNK), jnp.float32),
        pltpu.VMEM((CHUNK, HID), jnp.float32),
        pltpu.VMEM((CHUNK, HID), jnp.float32),
        pltpu.VMEM((ROWS_PER_TILE,), jnp.float32),
        pltpu.VMEM_SHARED((NPAD, HID), jnp.float32),
        pltpu.VMEM_SHARED((NPAD,), jnp.float32),
        pltpu.VMEM_SHARED((NPAD,), jnp.float32),
        pltpu.VMEM_SHARED((NPAD,), jnp.float32),
        pltpu.SemaphoreType.DMA,
        pltpu.SemaphoreType.DMA,
        pltpu.SemaphoreType.DMA,
        pltpu.SemaphoreType.DMA,
        pltpu.SemaphoreType.DMA,
    ],
)
def _sc_prop1(asrc_hbm, adst_hbm, shift_hbm, src2_hbm, dst2_hbm, xp_hbm,
              s1_out, den_out, ex_out,
              asrc_v, adst_v, shift_v, src2_v, dst2_v, ex2_v,
              rows_a, rows_b, zd_v, s1_sh, den_sh, as_sh, ad_sh,
              sem_a, sem_b, sem_sa, sem_sb, sem_d):
    cid = lax.axis_index("c")
    sid = lax.axis_index("s")

    @pl.when(sid == 0)
    def _():
        pltpu.sync_copy(asrc_hbm, as_sh)
        pltpu.sync_copy(adst_hbm, ad_sh)
    pltpu.sync_copy(shift_hbm, shift_v)

    def _zrows(i, c):
        for j in range(4):
            rows_a[i, pl.ds(j * 16, 16)] = _zero16()
        return c
    lax.fori_loop(0, CHUNK, _zrows, 0)

    def _zd(i, c):
        zd_v[pl.ds(pl.multiple_of(i * 16, 16), 16)] = _zero16()
        return c
    lax.fori_loop(0, ROWS_PER_TILE // 16, _zd, 0)

    r0 = pl.multiple_of(sid * ROWS_PER_TILE, ROWS_PER_TILE)
    for k in range(ROWS_PER_TILE // CHUNK):
        pltpu.sync_copy(rows_a, s1_sh.at[pl.ds(r0 + k * CHUNK, CHUNK)])
    pltpu.sync_copy(zd_v, den_sh.at[pl.ds(r0, ROWS_PER_TILE)])
    plsc.subcore_barrier()
    pltpu.sync_copy(as_sh, asrc_v)
    pltpu.sync_copy(ad_sh, adst_v)

    shv = shift_v[...]

    def _exs(c):
        for g in range(8):
            si = src2_v[c, pl.ds(g * 16, 16)]
            di = dst2_v[c, pl.ds(g * 16, 16)]
            a = plsc.load_gather(asrc_v, [si]) + plsc.load_gather(adst_v, [di])
            a = jnp.where(a >= 0.0, a, 0.2 * a)
            ex2_v[c, pl.ds(g * 16, 16)] = jnp.exp(a - shv)

    def _scale(buf, c):
        def _mul(g, cc):
            exg = ex2_v[c, pl.ds(pl.multiple_of(g * 16, 16), 16)]
            for l in range(16):
                sc = exg[l]
                e = g * 16 + l
                for j in range(4):
                    buf[e, pl.ds(j * 16, 16)] = buf[e, pl.ds(j * 16, 16)] * sc
            return cc
        lax.fori_loop(0, CHUNK // 16, _mul, 0)

    def _run(nc, crow0):
        pltpu.sync_copy(src2_hbm.at[pl.ds(crow0, nc)],
                        src2_v.at[pl.ds(0, nc)])
        pltpu.sync_copy(dst2_hbm.at[pl.ds(crow0, nc)],
                        dst2_v.at[pl.ds(0, nc)])
        pltpu.async_copy(xp_hbm.at[src2_v.at[0]], rows_a, sem_a)
        pltpu.async_copy(xp_hbm.at[src2_v.at[1]], rows_b, sem_b)

        def _pipe(i, carry):
            c0 = pl.multiple_of(i * 2, 2)
            c1 = c0 + 1
            _exs(c0)
            pltpu.make_async_copy(xp_hbm.at[src2_v.at[c0]], rows_a,
                                  sem_a).wait()
            _scale(rows_a, c0)
            pltpu.async_copy(rows_a, s1_sh.at[dst2_v.at[c0]], sem_sa,
                             add=True)
            pltpu.async_copy(ex2_v.at[c0], den_sh.at[dst2_v.at[c0]], sem_d,
                             add=True)
            _exs(c1)
            pltpu.make_async_copy(xp_hbm.at[src2_v.at[c1]], rows_b,
                                  sem_b).wait()
            _scale(rows_b, c1)
            pltpu.async_copy(rows_b, s1_sh.at[dst2_v.at[c1]], sem_sb,
                             add=True)
            pltpu.async_copy(ex2_v.at[c1], den_sh.at[dst2_v.at[c1]], sem_d,
                             add=True)
            pltpu.make_async_copy(rows_a, s1_sh.at[dst2_v.at[c0]],
                                  sem_sa).wait()
            pltpu.async_copy(xp_hbm.at[src2_v.at[c0 + 2]], rows_a, sem_a)

            @pl.when(i < (nc - 1) // 2 - 1)
            def _():
                pltpu.make_async_copy(rows_b, s1_sh.at[dst2_v.at[c1]],
                                      sem_sb).wait()
                pltpu.async_copy(xp_hbm.at[src2_v.at[c1 + 2]], rows_b, sem_b)
            return carry
        lax.fori_loop(0, (nc - 1) // 2, _pipe, 0)

        last = nc - 1
        _exs(last)
        pltpu.make_async_copy(rows_b, s1_sh.at[dst2_v.at[last - 1]],
                              sem_sb).wait()
        pltpu.make_async_copy(xp_hbm.at[src2_v.at[last]], rows_a,
                              sem_a).wait()
        _scale(rows_a, last)
        pltpu.sync_copy(rows_a, s1_sh.at[dst2_v.at[last]], add=True)
        pltpu.async_copy(ex2_v.at[last], den_sh.at[dst2_v.at[last]], sem_d,
                         add=True)

        pltpu.sync_copy(ex2_v.at[pl.ds(0, nc)],
                        ex_out.at[pl.ds(crow0, nc)])

        def _dend(c, carry):
            pltpu.make_async_copy(ex2_v.at[c], den_sh.at[dst2_v.at[c]],
                                  sem_d).wait()
            return carry
        lax.fori_loop(0, nc, _dend, 0)

    @pl.when(cid == 0)
    def _():
        _run(CA1, pl.multiple_of(sid * CA1, 1))

    @pl.when(cid == 1)
    def _():
        _run(CB1, pl.multiple_of(16 * CA1 + sid * CB1, 1))

    plsc.subcore_barrier()
    pltpu.sync_copy(s1_sh.at[pl.ds(r0, ROWS_PER_TILE)],
                    s1_out.at[cid, pl.ds(r0, ROWS_PER_TILE)])
    pltpu.sync_copy(den_sh.at[pl.ds(r0, ROWS_PER_TILE)],
                    den_out.at[cid, pl.ds(r0, ROWS_PER_TILE)])


@functools.partial(
    pl.kernel,
    out_type=[
        jax.ShapeDtypeStruct((2, NPAD, LAT), jnp.float32),
    ],
    mesh=_SC_MESH,
    compiler_params=pltpu.CompilerParams(needs_layout_passes=False, use_tc_tiling_on_sc=False),
    scratch_types=[
        pltpu.VMEM((CA2, CHUNK), jnp.int32),
        pltpu.VMEM((CA2, CHUNK), jnp.int32),
        pltpu.VMEM((CA2, CHUNK), jnp.float32),
        pltpu.VMEM((CHUNK, LAT), jnp.float32),
        pltpu.VMEM((CHUNK, LAT), jnp.float32),
        pltpu.VMEM_SHARED((NPAD, LAT), jnp.float32),
        pltpu.SemaphoreType.DMA,
        pltpu.SemaphoreType.DMA,
        pltpu.SemaphoreType.DMA,
        pltpu.SemaphoreType.DMA,
    ],
)
def _sc_prop2(src2_hbm, dst2_hbm, ex2_hbm, lat_hbm,
              s2_out,
              src2_v, dst2_v, ex2_v, rows_a, rows_b, s2_sh,
              sem_a, sem_b, sem_sa, sem_sb):
    cid = lax.axis_index("c")
    sid = lax.axis_index("s")

    def _zrows(i, c):
        rows_a[i, pl.ds(0, 16)] = _zero16()
        return c
    lax.fori_loop(0, CHUNK, _zrows, 0)

    r0 = pl.multiple_of(sid * ROWS_PER_TILE, ROWS_PER_TILE)
    for k in range(ROWS_PER_TILE // CHUNK):
        pltpu.sync_copy(rows_a, s2_sh.at[pl.ds(r0 + k * CHUNK, CHUNK)])
    plsc.subcore_barrier()

    def _scale(buf, c):
        def _mul(g, cc):
            exg = ex2_v[c, pl.ds(pl.multiple_of(g * 16, 16), 16)]
            for l in range(16):
                e = g * 16 + l
                buf[e, pl.ds(0, 16)] = buf[e, pl.ds(0, 16)] * exg[l]
            return cc
        lax.fori_loop(0, CHUNK // 16, _mul, 0)

    def _run(nc, crow0):
        pltpu.sync_copy(src2_hbm.at[pl.ds(crow0, nc)],
                        src2_v.at[pl.ds(0, nc)])
        pltpu.sync_copy(dst2_hbm.at[pl.ds(crow0, nc)],
                        dst2_v.at[pl.ds(0, nc)])
        pltpu.sync_copy(ex2_hbm.at[pl.ds(crow0, nc)],
                        ex2_v.at[pl.ds(0, nc)])
        pltpu.async_copy(lat_hbm.at[src2_v.at[0]], rows_a, sem_a)
        pltpu.async_copy(lat_hbm.at[src2_v.at[1]], rows_b, sem_b)

        def _pipe(i, carry):
            c0 = pl.multiple_of(i * 2, 2)
            c1 = c0 + 1
            pltpu.make_async_copy(lat_hbm.at[src2_v.at[c0]], rows_a,
                                  sem_a).wait()
            _scale(rows_a, c0)
            pltpu.async_copy(rows_a, s2_sh.at[dst2_v.at[c0]], sem_sa,
                             add=True)
            pltpu.make_async_copy(lat_hbm.at[src2_v.at[c1]], rows_b,
                                  sem_b).wait()
            _scale(rows_b, c1)
            pltpu.async_copy(rows_b, s2_sh.at[dst2_v.at[c1]], sem_sb,
                             add=True)
            pltpu.make_async_copy(rows_a, s2_sh.at[dst2_v.at[c0]],
                                  sem_sa).wait()
            pltpu.async_copy(lat_hbm.at[src2_v.at[c0 + 2]], rows_a, sem_a)

            @pl.when(i < (nc - 1) // 2 - 1)
            def _():
                pltpu.make_async_copy(rows_b, s2_sh.at[dst2_v.at[c1]],
                                      sem_sb).wait()
                pltpu.async_copy(lat_hbm.at[src2_v.at[c1 + 2]], rows_b, sem_b)
            return carry
        lax.fori_loop(0, (nc - 1) // 2, _pipe, 0)

        last = nc - 1
        pltpu.make_async_copy(rows_b, s2_sh.at[dst2_v.at[last - 1]],
                              sem_sb).wait()
        pltpu.make_async_copy(lat_hbm.at[src2_v.at[last]], rows_a,
                              sem_a).wait()
        _scale(rows_a, last)
        pltpu.sync_copy(rows_a, s2_sh.at[dst2_v.at[last]], add=True)

    @pl.when(cid == 0)
    def _():
        _run(CA2, pl.multiple_of(sid * CA2, 1))

    @pl.when(cid == 1)
    def _():
        _run(CB2, pl.multiple_of(16 * CA2 + sid * CB2, 1))

    plsc.subcore_barrier()
    pltpu.sync_copy(s2_sh.at[pl.ds(r0, ROWS_PER_TILE)],
                    s2_out.at[cid, pl.ds(r0, ROWS_PER_TILE)])


def kernel(x, edge_index, W1, W2, att_src1, att_dst1):
    asv = att_src1.reshape(1, HID)
    adv = att_dst1.reshape(1, HID)

    xp, asrc, adst, shift = pl.pallas_call(
        _tc1_body,
        out_shape=[
            jax.ShapeDtypeStruct((N, HID), jnp.float32),
            jax.ShapeDtypeStruct((NPAD, 1), jnp.float32),
            jax.ShapeDtypeStruct((NPAD, 1), jnp.float32),
            jax.ShapeDtypeStruct((1, 128), jnp.float32),
        ],
    )(x, W1, asv, adv)

    pad = EPAD - E
    src2 = jnp.concatenate([edge_index[0],
                            jnp.zeros((pad,), jnp.int32)]).reshape(-1, CHUNK)
    dst2 = jnp.concatenate([edge_index[1],
                            N + (jnp.arange(pad, dtype=jnp.int32) % 8)]
                           ).reshape(-1, CHUNK)

    s1p, denp, ex2 = _sc_prop1(asrc.reshape(NPAD), adst.reshape(NPAD),
                               shift[0, :16], src2, dst2, xp)

    den3 = denp.reshape(2, NPAD, 1)
    latent = pl.pallas_call(
        _tc2_body,
        out_shape=jax.ShapeDtypeStruct((N, LAT), jnp.float32),
    )(s1p, den3, W2)

    s2p = _sc_prop2(src2, dst2, ex2, latent)
    if isinstance(s2p, (list, tuple)):
        s2p = s2p[0]

    recon = pl.pallas_call(
        _tc3_body,
        out_shape=jax.ShapeDtypeStruct((N, DIN), jnp.float32),
    )(s2p, den3, W2, W1)

    return latent, recon
